# baseline (speedup 1.0000x reference)
import jax
import jax.numpy as jnp
from jax.experimental import pallas as pl
from jax.experimental.pallas import tpu as pltpu

_B, _C, _H, _W = 16, 256, 32, 32


def _body(row_ref, col_ref, out_ref, scratch, sems):
    col_t = col_ref[...].T
    row_t = row_ref[...].T
    scratch[:_C] = jnp.broadcast_to(col_t[:, None, :], (_C, _H, _W))
    scratch[_C:] = jnp.broadcast_to(row_t[:, :, None], (_C, _H, _W))
    for b in range(_B):
        pltpu.make_async_copy(scratch, out_ref.at[b], sems.at[b]).start()
    for b in range(_B):
        pltpu.make_async_copy(scratch, out_ref.at[b], sems.at[b]).wait()


def kernel(mask, row_embed, col_embed):
    b = mask.shape[0]
    h, w = mask.shape[-2], mask.shape[-1]
    c = row_embed.shape[-1]
    return pl.pallas_call(
        _body,
        grid=(1,),
        in_specs=[
            pl.BlockSpec((h, c), lambda i: (0, 0)),
            pl.BlockSpec((w, c), lambda i: (0, 0)),
        ],
        out_specs=pl.BlockSpec(memory_space=pl.ANY),
        out_shape=jax.ShapeDtypeStruct((b, 2 * c, h, w), jnp.float32),
        scratch_shapes=[
            pltpu.VMEM((2 * c, h, w), jnp.float32),
            pltpu.SemaphoreType.DMA((b,)),
        ],
    )(row_embed, col_embed)


# speedup vs baseline: 2.7838x; 2.7838x over previous
import jax
import jax.numpy as jnp
from jax.experimental import pallas as pl
from jax.experimental.pallas import tpu as pltpu

_B, _C, _H, _W = 16, 256, 32, 32
_SUB = (_H * _W) // 128


def _body(row_ref, col_ref, out_ref, scratch, sems):
    dn = (((0,), (0,)), ((), ()))
    iota_t = jax.lax.broadcasted_iota(jnp.int32, (_W, 128), 0)
    iota_l = jax.lax.broadcasted_iota(jnp.int32, (_W, 128), 1)
    sel_x = ((iota_l & (_W - 1)) == iota_t).astype(jnp.float32)
    col_t4 = jax.lax.dot_general(
        col_ref[...], sel_x, dn, preferred_element_type=jnp.float32)
    scratch[:_C] = jnp.broadcast_to(col_t4[:, None, :], (_C, _SUB, 128))
    for s in range(_SUB):
        sel_y = ((iota_l >> 5) + 4 * s == iota_t).astype(jnp.float32)
        scratch[_C:, s] = jax.lax.dot_general(
            row_ref[...], sel_y, dn, preferred_element_type=jnp.float32)

    for b in range(_B):
        pltpu.make_async_copy(scratch, out_ref.at[b], sems.at[b]).start()
    for b in range(_B):
        pltpu.make_async_copy(scratch, out_ref.at[b], sems.at[b]).wait()


def kernel(mask, row_embed, col_embed):
    b = mask.shape[0]
    h, w = mask.shape[-2], mask.shape[-1]
    c = row_embed.shape[-1]
    out = pl.pallas_call(
        _body,
        grid=(1,),
        in_specs=[
            pl.BlockSpec((h, c), lambda i: (0, 0)),
            pl.BlockSpec((w, c), lambda i: (0, 0)),
        ],
        out_specs=pl.BlockSpec(memory_space=pl.ANY),
        out_shape=jax.ShapeDtypeStruct((b, 2 * c, _SUB, 128), jnp.float32),
        scratch_shapes=[
            pltpu.VMEM((2 * c, _SUB, 128), jnp.float32),
            pltpu.SemaphoreType.DMA((b,)),
        ],
    )(row_embed, col_embed)
    return out.reshape(b, 2 * c, h, w)


# speedup vs baseline: 10.2646x; 3.6873x over previous
import jax
import jax.numpy as jnp
from jax.experimental import pallas as pl
from jax.experimental.pallas import tpu as pltpu

_B, _C, _H, _W = 16, 256, 32, 32


def _body(row_ref, col_ref, out_ref, scratch, sems):
    scratch[:, :, :_C] = jnp.broadcast_to(col_ref[...][None, :, :], (_H, _W, _C))
    scratch[:, :, _C:] = jnp.broadcast_to(row_ref[...][:, None, :], (_H, _W, _C))
    for b in range(_B):
        pltpu.make_async_copy(scratch, out_ref.at[b], sems.at[b]).start()
    for b in range(_B):
        pltpu.make_async_copy(scratch, out_ref.at[b], sems.at[b]).wait()


def kernel(mask, row_embed, col_embed):
    b = mask.shape[0]
    h, w = mask.shape[-2], mask.shape[-1]
    c = row_embed.shape[-1]
    out = pl.pallas_call(
        _body,
        grid=(1,),
        in_specs=[
            pl.BlockSpec((h, c), lambda i: (0, 0)),
            pl.BlockSpec((w, c), lambda i: (0, 0)),
        ],
        out_specs=pl.BlockSpec(memory_space=pl.ANY),
        out_shape=jax.ShapeDtypeStruct((b, h, w, 2 * c), jnp.float32),
        scratch_shapes=[
            pltpu.VMEM((h, w, 2 * c), jnp.float32),
            pltpu.SemaphoreType.DMA((b,)),
        ],
    )(row_embed, col_embed)
    return jnp.transpose(out, (0, 3, 1, 2))
